# Initial kernel scaffold; baseline (speedup 1.0000x reference)
import jax
import jax.numpy as jnp
from jax.experimental import pallas as pl


def kernel(x, node_in, node_out, relation, edge_weight, W0_lin, b0_lin, W0_self, b0_self, W1_lin, b1_lin, W1_self, b1_self):
    raise NotImplementedError("write your pallas kernel here")


# speedup vs baseline: 1.1651x; 1.1651x over previous
import jax
import jax.numpy as jnp
from jax.experimental import pallas as pl

N = 10000
R = 4
D = 128
BLK = 1000


def _layer0_body(upd_ref, x_ref, wl_ref, ws_ref, b_ref, o_ref):
    acc = jnp.dot(upd_ref[...], wl_ref[...], preferred_element_type=jnp.float32)
    acc += jnp.dot(x_ref[...], ws_ref[...], preferred_element_type=jnp.float32)
    o_ref[...] = jnp.maximum(acc + b_ref[...], 0.0)


def _layer1_body(upd_ref, h_ref, wl_ref, ws_ref, b_ref, o_ref):
    acc = jnp.dot(upd_ref[0], wl_ref[...], preferred_element_type=jnp.float32)
    acc += jnp.dot(h_ref[...], ws_ref[...], preferred_element_type=jnp.float32)
    o_ref[0] = jnp.maximum(acc + b_ref[...], 0.0)


def kernel(x, node_in, node_out, relation, edge_weight,
           W0_lin, b0_lin, W0_self, b0_self,
           W1_lin, b1_lin, W1_self, b1_self):
    idx = node_out * R + relation
    deg = jax.ops.segment_sum(edge_weight, idx, num_segments=N * R)
    ew = edge_weight / deg[idx]

    msg0 = ew[:, None] * x[node_in]
    upd0 = jax.ops.segment_sum(msg0, idx, num_segments=N * R).reshape(N, R * D)

    b0 = (b0_lin + b0_self).reshape(1, D)
    h = pl.pallas_call(
        _layer0_body,
        grid=(N // BLK,),
        in_specs=[
            pl.BlockSpec((BLK, R * D), lambda i: (i, 0)),
            pl.BlockSpec((BLK, D), lambda i: (i, 0)),
            pl.BlockSpec((R * D, D), lambda i: (0, 0)),
            pl.BlockSpec((D, D), lambda i: (0, 0)),
            pl.BlockSpec((1, D), lambda i: (0, 0)),
        ],
        out_specs=pl.BlockSpec((BLK, D), lambda i: (i, 0)),
        out_shape=jax.ShapeDtypeStruct((N, D), jnp.float32),
    )(upd0, x, W0_lin, W0_self, b0)

    msg1 = ew[:, None] * h[node_in]
    upd1 = jax.ops.segment_sum(msg1, idx, num_segments=N * R)
    upd1 = upd1.reshape(N, R, D).transpose(1, 0, 2)

    b1 = (b1_lin + b1_self).reshape(1, D)
    out = pl.pallas_call(
        _layer1_body,
        grid=(R, N // BLK),
        in_specs=[
            pl.BlockSpec((1, BLK, D), lambda r, i: (r, i, 0)),
            pl.BlockSpec((BLK, D), lambda r, i: (i, 0)),
            pl.BlockSpec((D, D), lambda r, i: (0, 0)),
            pl.BlockSpec((D, D), lambda r, i: (0, 0)),
            pl.BlockSpec((1, D), lambda r, i: (0, 0)),
        ],
        out_specs=pl.BlockSpec((1, BLK, D), lambda r, i: (r, i, 0)),
        out_shape=jax.ShapeDtypeStruct((R, N, D), jnp.float32),
    )(upd1, h, W1_lin, W1_self, b1)

    return out
